# baseline (speedup 1.0000x reference)
import jax
import jax.numpy as jnp
from jax.experimental import pallas as pl
from jax.experimental.pallas import tpu as pltpu
import jax.experimental.pallas.tpu_sc as plsc

_N, _D = 100000, 128
_SC_BLOCK = 200

_vector_mesh = plsc.VectorSubcoreMesh(
    core_axis_name="core", subcore_axis_name="subcore"
)


def kernel(data_list):
    @pl.kernel(
        out_type=jax.ShapeDtypeStruct((_N, _D), jnp.float32),
        mesh=_vector_mesh,
    )
    def _sc_copy(x_hbm, o_hbm):
        def body(in_vmem, out_vmem):
            out_vmem[...] = in_vmem[...]

        pltpu.emit_pipeline(
            body,
            grid=(_N // _SC_BLOCK,),
            in_specs=[pl.BlockSpec((_SC_BLOCK, _D), lambda i: (i, 0))],
            out_specs=[pl.BlockSpec((_SC_BLOCK, _D), lambda i: (i, 0))],
            core_axis_name=("core", "subcore"),
            dimension_semantics=(pltpu.PARALLEL,),
        )(x_hbm, o_hbm)

    return _sc_copy(data_list)


# speedup vs baseline: 5.0102x; 5.0102x over previous
import jax
import jax.numpy as jnp
from jax.experimental import pallas as pl
from jax.experimental.pallas import tpu as pltpu

_N, _D = 100000, 128
_NCH = 50
_CH = _N // _NCH


def _dma_pipeline(x_ref, o_ref, buf, in_sems, out_sems):
    for i in range(_NCH):
        pltpu.make_async_copy(
            x_ref.at[pl.ds(i * _CH, _CH)], buf.at[i], in_sems.at[i]
        ).start()
    for i in range(_NCH):
        pltpu.make_async_copy(
            x_ref.at[pl.ds(i * _CH, _CH)], buf.at[i], in_sems.at[i]
        ).wait()
        pltpu.make_async_copy(
            buf.at[i], o_ref.at[pl.ds(i * _CH, _CH)], out_sems.at[i]
        ).start()
    for i in range(_NCH):
        pltpu.make_async_copy(
            buf.at[i], o_ref.at[pl.ds(i * _CH, _CH)], out_sems.at[i]
        ).wait()


def kernel(data_list):
    return pl.pallas_call(
        _dma_pipeline,
        in_specs=[pl.BlockSpec(memory_space=pltpu.MemorySpace.HBM)],
        out_specs=pl.BlockSpec(memory_space=pltpu.MemorySpace.HBM),
        out_shape=jax.ShapeDtypeStruct((_N, _D), jnp.float32),
        scratch_shapes=[
            pltpu.VMEM((_NCH, _CH, _D), jnp.float32),
            pltpu.SemaphoreType.DMA((_NCH,)),
            pltpu.SemaphoreType.DMA((_NCH,)),
        ],
        compiler_params=pltpu.CompilerParams(vmem_limit_bytes=60 * 2**20),
    )(data_list)


# speedup vs baseline: 5.0526x; 1.0085x over previous
import jax
import jax.numpy as jnp
from jax.experimental import pallas as pl
from jax.experimental.pallas import tpu as pltpu

_N, _D = 100000, 128
_NCH = 20
_CH = _N // _NCH


def _dma_pipeline(x_ref, o_ref, buf, in_sems, out_sems):
    for i in range(_NCH):
        pltpu.make_async_copy(
            x_ref.at[pl.ds(i * _CH, _CH)], buf.at[i], in_sems.at[i]
        ).start()
    for i in range(_NCH):
        pltpu.make_async_copy(
            x_ref.at[pl.ds(i * _CH, _CH)], buf.at[i], in_sems.at[i]
        ).wait()
        pltpu.make_async_copy(
            buf.at[i], o_ref.at[pl.ds(i * _CH, _CH)], out_sems.at[i]
        ).start()
    for i in range(_NCH):
        pltpu.make_async_copy(
            buf.at[i], o_ref.at[pl.ds(i * _CH, _CH)], out_sems.at[i]
        ).wait()


def kernel(data_list):
    return pl.pallas_call(
        _dma_pipeline,
        in_specs=[pl.BlockSpec(memory_space=pltpu.MemorySpace.HBM)],
        out_specs=pl.BlockSpec(memory_space=pltpu.MemorySpace.HBM),
        out_shape=jax.ShapeDtypeStruct((_N, _D), jnp.float32),
        scratch_shapes=[
            pltpu.VMEM((_NCH, _CH, _D), jnp.float32),
            pltpu.SemaphoreType.DMA((_NCH,)),
            pltpu.SemaphoreType.DMA((_NCH,)),
        ],
        compiler_params=pltpu.CompilerParams(vmem_limit_bytes=60 * 2**20),
    )(data_list)


# speedup vs baseline: 5.0826x; 1.0059x over previous
import jax
import jax.numpy as jnp
from jax.experimental import pallas as pl
from jax.experimental.pallas import tpu as pltpu

_N, _D = 100000, 128
_NCH = 10
_CH = _N // _NCH


def _dma_pipeline(x_ref, o_ref, buf, in_sems, out_sems):
    for i in range(_NCH):
        pltpu.make_async_copy(
            x_ref.at[pl.ds(i * _CH, _CH)], buf.at[i], in_sems.at[i]
        ).start()
    for i in range(_NCH):
        pltpu.make_async_copy(
            x_ref.at[pl.ds(i * _CH, _CH)], buf.at[i], in_sems.at[i]
        ).wait()
        pltpu.make_async_copy(
            buf.at[i], o_ref.at[pl.ds(i * _CH, _CH)], out_sems.at[i]
        ).start()
    for i in range(_NCH):
        pltpu.make_async_copy(
            buf.at[i], o_ref.at[pl.ds(i * _CH, _CH)], out_sems.at[i]
        ).wait()


def kernel(data_list):
    return pl.pallas_call(
        _dma_pipeline,
        in_specs=[pl.BlockSpec(memory_space=pltpu.MemorySpace.HBM)],
        out_specs=pl.BlockSpec(memory_space=pltpu.MemorySpace.HBM),
        out_shape=jax.ShapeDtypeStruct((_N, _D), jnp.float32),
        scratch_shapes=[
            pltpu.VMEM((_NCH, _CH, _D), jnp.float32),
            pltpu.SemaphoreType.DMA((_NCH,)),
            pltpu.SemaphoreType.DMA((_NCH,)),
        ],
        compiler_params=pltpu.CompilerParams(vmem_limit_bytes=60 * 2**20),
    )(data_list)


# speedup vs baseline: 5.2349x; 1.0300x over previous
import jax
import jax.numpy as jnp
from jax.experimental import pallas as pl
from jax.experimental.pallas import tpu as pltpu

_N, _D = 100000, 128
_CHUNKS = (800, 1200, 2000, 4000, 8000,
           13600, 13600, 13600, 13600, 13600,
           8000, 4000, 2000, 1200, 800)
_OFFS = tuple(sum(_CHUNKS[:i]) for i in range(len(_CHUNKS)))
_NCH = len(_CHUNKS)
assert sum(_CHUNKS) == _N


def _dma_pipeline(x_ref, o_ref, buf, in_sems, out_sems):
    for i in range(_NCH):
        pltpu.make_async_copy(
            x_ref.at[pl.ds(_OFFS[i], _CHUNKS[i])],
            buf.at[pl.ds(_OFFS[i], _CHUNKS[i])],
            in_sems.at[i],
        ).start()
    for i in range(_NCH):
        pltpu.make_async_copy(
            x_ref.at[pl.ds(_OFFS[i], _CHUNKS[i])],
            buf.at[pl.ds(_OFFS[i], _CHUNKS[i])],
            in_sems.at[i],
        ).wait()
        pltpu.make_async_copy(
            buf.at[pl.ds(_OFFS[i], _CHUNKS[i])],
            o_ref.at[pl.ds(_OFFS[i], _CHUNKS[i])],
            out_sems.at[i],
        ).start()
    for i in range(_NCH):
        pltpu.make_async_copy(
            buf.at[pl.ds(_OFFS[i], _CHUNKS[i])],
            o_ref.at[pl.ds(_OFFS[i], _CHUNKS[i])],
            out_sems.at[i],
        ).wait()


def kernel(data_list):
    return pl.pallas_call(
        _dma_pipeline,
        in_specs=[pl.BlockSpec(memory_space=pltpu.MemorySpace.HBM)],
        out_specs=pl.BlockSpec(memory_space=pltpu.MemorySpace.HBM),
        out_shape=jax.ShapeDtypeStruct((_N, _D), jnp.float32),
        scratch_shapes=[
            pltpu.VMEM((_N, _D), jnp.float32),
            pltpu.SemaphoreType.DMA((_NCH,)),
            pltpu.SemaphoreType.DMA((_NCH,)),
        ],
        compiler_params=pltpu.CompilerParams(vmem_limit_bytes=60 * 2**20),
    )(data_list)
